# Initial kernel scaffold; baseline (speedup 1.0000x reference)
import jax
import jax.numpy as jnp
from jax.experimental import pallas as pl


def kernel(xyz, points, W1, b1, g1, be1, W2, b2, g2, be2, W3, b3, g3, be3):
    raise NotImplementedError("write your pallas kernel here")


# speedup vs baseline: 1.7814x; 1.7814x over previous
import functools
import jax
import jax.numpy as jnp
import numpy as np
from jax import lax
from jax.experimental import pallas as pl
from jax.experimental.pallas import tpu as pltpu

S = 512
NS = 32
RAD4 = 0.4 ** 4
EPS = 1e-5


def _table_kernel(pts_ref, xyz_ref, w1pt_ref, w1xt_ref, b1_ref, out_ref):
    pts = pts_ref[...]
    z = jnp.dot(pts, w1pt_ref[...], preferred_element_type=jnp.float32)
    for c in range(3):
        z += xyz_ref[:, c:c + 1] * w1xt_ref[c:c + 1, :]
    out_ref[...] = z + b1_ref[...]


def _stats_kernel(z_ref, out_ref, acc):
    i = pl.program_id(0)

    @pl.when(i == 0)
    def _():
        acc[...] = jnp.zeros_like(acc)

    z = z_ref[...]
    acc[0:1, :] += jnp.sum(z, axis=0, keepdims=True)
    acc[1:2, :] += jnp.sum(z * z, axis=0, keepdims=True)

    @pl.when(i == pl.num_programs(0) - 1)
    def _():
        out_ref[...] = acc[...]


def _layer2_kernel(z_ref, sc_ref, sh_ref, w2t_ref, out_ref, st_ref, acc):
    i = pl.program_id(0)

    @pl.when(i == 0)
    def _():
        acc[...] = jnp.zeros_like(acc)

    a = jnp.maximum(z_ref[...] * sc_ref[...] + sh_ref[...], 0.0)
    z2 = jnp.dot(a, w2t_ref[...], preferred_element_type=jnp.float32)
    out_ref[...] = z2
    acc[0:1, :] += jnp.sum(z2, axis=0, keepdims=True)
    acc[1:2, :] += jnp.sum(z2 * z2, axis=0, keepdims=True)

    @pl.when(i == pl.num_programs(0) - 1)
    def _():
        st_ref[...] = acc[...]


def _layer3_kernel(z_ref, sc_ref, sh_ref, w3t_ref, out_ref, st_ref, acc):
    i = pl.program_id(0)

    @pl.when(i == 0)
    def _():
        acc[...] = jnp.zeros_like(acc)

    a = jnp.maximum(z_ref[...] * sc_ref[...] + sh_ref[...], 0.0)
    z3 = jnp.dot(a, w3t_ref[...], preferred_element_type=jnp.float32)
    acc[0:1, :] += jnp.sum(z3, axis=0, keepdims=True)
    acc[1:2, :] += jnp.sum(z3 * z3, axis=0, keepdims=True)
    blk = z3.shape[0]
    out_ref[...] = jnp.max(z3.reshape(blk // NS, NS, z3.shape[1]), axis=1)

    @pl.when(i == pl.num_programs(0) - 1)
    def _():
        st_ref[...] = acc[...]


def _final_kernel(z_ref, sc_ref, sh_ref, out_ref):
    out_ref[...] = jnp.maximum(z_ref[...] * sc_ref[...] + sh_ref[...], 0.0)


def _row_blocked(nrows, blk, ncols):
    return pl.BlockSpec((blk, ncols), lambda i: (i, 0))


def _bcast_spec(shape):
    return pl.BlockSpec(shape, lambda i: (0, 0))


def _scale_shift(sums, ntot, g, be):
    m = sums[0] / ntot
    var = jnp.maximum(sums[1] / ntot - m * m, 0.0)
    sc = g / jnp.sqrt(var + EPS)
    return sc, be - m * sc


def _fps_xla(xyzT, npoint):
    Bn, N, _ = xyzT.shape

    def body(carry, _):
        distance, farthest = carry
        centroid = xyzT[jnp.arange(Bn), farthest][:, None, :]
        dist = jnp.sum((xyzT - centroid) ** 2, axis=-1)
        distance = jnp.minimum(distance, dist)
        nxt = jnp.argmax(distance, axis=-1).astype(jnp.int32)
        return (distance, nxt), farthest

    init = (jnp.full((Bn, N), 1e10, dtype=xyzT.dtype),
            jnp.zeros((Bn,), dtype=jnp.int32))
    _, cent = jax.lax.scan(body, init, None, length=npoint)
    return jnp.transpose(cent)


def _ball_query_xla(xyzT, new_xyz):
    Bn, N, _ = xyzT.shape
    q2 = jnp.sum(new_xyz ** 2, axis=-1)[..., None]
    x2 = jnp.sum(xyzT ** 2, axis=-1)[:, None, :]
    d2 = jnp.clip(q2 + x2 - 2.0 * jnp.einsum('bsc,bnc->bsn', new_xyz, xyzT),
                  0.0, None)
    nearest = jnp.argmin(d2, axis=-1).astype(jnp.int32)
    inball = d2 < RAD4
    order = jnp.where(inball, jnp.arange(N, dtype=jnp.int32)[None, None, :],
                      jnp.int32(N + 1))
    sortidx = jnp.sort(order, axis=-1)[:, :, :NS]
    return jnp.where(sortidx <= N, sortidx, nearest[:, :, None]).astype(jnp.int32)


def kernel(xyz, points, W1, b1, g1, be1, W2, b2, g2, be2, W3, b3, g3, be3):
    B, _, N = xyz.shape
    D = points.shape[1]
    C2 = W2.shape[0]
    C3 = W3.shape[0]
    xyzT = jnp.transpose(xyz, (0, 2, 1))
    ptsT = jnp.transpose(points, (0, 2, 1))
    P = B * S * NS
    ntot = jnp.float32(P)

    BLK0 = 2048
    table = pl.pallas_call(
        _table_kernel,
        grid=(B * N // BLK0,),
        in_specs=[_row_blocked(B * N, BLK0, D),
                  pl.BlockSpec((BLK0, 3), lambda i: (i, 0)),
                  _bcast_spec((D, D)),
                  _bcast_spec((3, D)),
                  pl.BlockSpec((1, D), lambda i: (0, 0))],
        out_specs=_row_blocked(B * N, BLK0, D),
        out_shape=jax.ShapeDtypeStruct((B * N, D), jnp.float32),
    )(ptsT.reshape(B * N, D), xyzT.reshape(B * N, 3),
      W1[:, 3:].T, W1[:, :3].T, b1[None, :])

    cent = _fps_xla(xyzT, S)
    new_xyz = jnp.take_along_axis(
        xyzT, cent[:, :, None].astype(jnp.int32), axis=1)
    idx = _ball_query_xla(xyzT, new_xyz)

    gidx = (idx + (jnp.arange(B, dtype=jnp.int32) * N)[:, None, None])
    gz = table[gidx.reshape(-1)]
    qcorr = jnp.einsum('bsc,dc->bsd', new_xyz, W1[:, :3])
    z1 = (gz.reshape(B, S, NS, D) - qcorr[:, :, None, :]).reshape(P, D)

    BLK = 1024
    grid = (P // BLK,)
    stats1 = pl.pallas_call(
        _stats_kernel,
        grid=grid,
        in_specs=[_row_blocked(P, BLK, D)],
        out_specs=pl.BlockSpec((8, D), lambda i: (0, 0)),
        out_shape=jax.ShapeDtypeStruct((8, D), jnp.float32),
        scratch_shapes=[pltpu.VMEM((8, D), jnp.float32)],
    )(z1)
    sc1, sh1 = _scale_shift(stats1, ntot, g1, be1)

    z2, stats2 = pl.pallas_call(
        _layer2_kernel,
        grid=grid,
        in_specs=[_row_blocked(P, BLK, D),
                  _bcast_spec((1, D)), _bcast_spec((1, D)),
                  _bcast_spec((D, C2))],
        out_specs=[_row_blocked(P, BLK, C2),
                   pl.BlockSpec((8, C2), lambda i: (0, 0))],
        out_shape=[jax.ShapeDtypeStruct((P, C2), jnp.float32),
                   jax.ShapeDtypeStruct((8, C2), jnp.float32)],
        scratch_shapes=[pltpu.VMEM((8, C2), jnp.float32)],
    )(z1, sc1[None, :], sh1[None, :], W2.T)
    sc2, sh2 = _scale_shift(stats2, ntot, g2, be2)

    zmax, stats3 = pl.pallas_call(
        _layer3_kernel,
        grid=grid,
        in_specs=[_row_blocked(P, BLK, C2),
                  _bcast_spec((1, C2)), _bcast_spec((1, C2)),
                  _bcast_spec((C2, C3))],
        out_specs=[pl.BlockSpec((BLK // NS, C3), lambda i: (i, 0)),
                   pl.BlockSpec((8, C3), lambda i: (0, 0))],
        out_shape=[jax.ShapeDtypeStruct((B * S, C3), jnp.float32),
                   jax.ShapeDtypeStruct((8, C3), jnp.float32)],
        scratch_shapes=[pltpu.VMEM((8, C3), jnp.float32)],
    )(z2, sc2[None, :], sh2[None, :], W3.T)
    sc3, sh3 = _scale_shift(stats3, ntot, g3, be3)

    out = pl.pallas_call(
        _final_kernel,
        grid=(B * S // BLK,),
        in_specs=[_row_blocked(B * S, BLK, C3),
                  _bcast_spec((1, C3)), _bcast_spec((1, C3))],
        out_specs=_row_blocked(B * S, BLK, C3),
        out_shape=jax.ShapeDtypeStruct((B * S, C3), jnp.float32),
    )(zmax, sc3[None, :], sh3[None, :])

    new_xyz_out = jnp.transpose(new_xyz, (0, 2, 1))
    new_points_out = jnp.transpose(out.reshape(B, S, C3), (0, 2, 1))
    return (new_xyz_out, new_points_out)
